# initial kernel scaffold (unmeasured)
import jax
import jax.numpy as jnp
from jax import lax
from jax.experimental import pallas as pl
from jax.experimental.pallas import tpu as pltpu

N_DEV = 32


def kernel(x, Win0, Wout0, Win1, Wout1, Win2, Wout2):
    blk, dmodel = x.shape
    B = N_DEV * blk
    CBLK = 1024

    def body(x_ref, win0_ref, wout0_ref, win1_ref, wout1_ref,
             win2_ref, wout2_ref, out_ref,
             xg_ref, pacc_ref, rs_ref, ag_ref, send_sems, recv_sems):
        d = lax.axis_index("i")
        right = lax.rem(d + 1, N_DEV)

        def rdma_hop(comm_ref, s, r):
            rdma = pltpu.make_async_remote_copy(
                src_ref=comm_ref.at[pl.ds(s * blk, blk), :],
                dst_ref=comm_ref.at[pl.ds(r * blk, blk), :],
                send_sem=send_sems.at[s],
                recv_sem=recv_sems.at[r],
                device_id=(right,),
                device_id_type=pl.DeviceIdType.MESH,
            )
            rdma.start()
            rdma.wait()

        def ring(n_hops, hop_fn):
            def pair(t, carry):
                hop_fn(2 * t, 0, 1)
                hop_fn(2 * t + 1, 1, 0)
                return carry
            lax.fori_loop(0, n_hops // 2, pair, 0)
            if n_hops % 2:
                h = n_hops - 1
                hop_fn(h, h % 2, (h + 1) % 2)

        def ag_hop(m):
            def fn(h, s, r):
                rdma_hop(ag_ref, s, r)
                origin = lax.rem(m - h - 1 + 2 * N_DEV, N_DEV)
                xg_ref[pl.ds(origin * blk, blk), :] = ag_ref[pl.ds(r * blk, blk), :]
            return fn

        def rs_hop(h, s, r):
            rdma_hop(rs_ref, s, r)
            c = lax.rem(d - h - 1 + 2 * N_DEV, N_DEV)
            cur = rs_ref[pl.ds(r * blk, blk), :]
            rs_ref[pl.ds(r * blk, blk), :] = cur + pacc_ref[pl.ds(c * blk, blk), :]

        def compute_layer(win_ref, wout_ref):
            win = win_ref[:].astype(jnp.bfloat16)
            wout = wout_ref[:].astype(jnp.bfloat16)
            for b0 in range(0, B, CBLK):
                xb = xg_ref[pl.ds(b0, CBLK), :]
                h = jnp.dot(xb, win, preferred_element_type=jnp.float32)
                h = jnp.maximum(h, 0.0).astype(jnp.bfloat16)
                pacc_ref[pl.ds(b0, CBLK), :] = jnp.dot(
                    h, wout, preferred_element_type=jnp.float32)

        myx = x_ref[:].astype(jnp.bfloat16)
        xg_ref[pl.ds(d * blk, blk), :] = myx
        ag_ref[pl.ds(0, blk), :] = myx
        ring(N_DEV - 1, ag_hop(d))

        for win_ref, wout_ref in ((win0_ref, wout0_ref),
                                  (win1_ref, wout1_ref),
                                  (win2_ref, wout2_ref)):
            compute_layer(win_ref, wout_ref)
            rs_ref[pl.ds(0, blk), :] = pacc_ref[pl.ds(d * blk, blk), :]
            ring(N_DEV - 1, rs_hop)
            m = right
            chunk = rs_ref[pl.ds(blk, blk), :].astype(jnp.bfloat16)
            xg_ref[pl.ds(m * blk, blk), :] = chunk
            ag_ref[pl.ds(0, blk), :] = chunk
            ring(N_DEV - 1, ag_hop(m))

        out_ref[:, :] = xg_ref[:, :].astype(jnp.float32)

    return pl.pallas_call(
        body,
        out_shape=jax.ShapeDtypeStruct((B, dmodel), jnp.float32),
        in_specs=[pl.BlockSpec(memory_space=pltpu.VMEM)] * 7,
        out_specs=pl.BlockSpec(memory_space=pltpu.VMEM),
        scratch_shapes=[
            pltpu.VMEM((B, dmodel), jnp.bfloat16),
            pltpu.VMEM((B, dmodel), jnp.float32),
            pltpu.VMEM((2 * blk, dmodel), jnp.float32),
            pltpu.VMEM((2 * blk, dmodel), jnp.bfloat16),
            pltpu.SemaphoreType.DMA((2,)),
            pltpu.SemaphoreType.DMA((2,)),
        ],
        compiler_params=pltpu.CompilerParams(collective_id=0),
    )(x, Win0, Wout0, Win1, Wout1, Win2, Wout2)


# baseline (device time: 882662 ns/iter reference)
import jax
import jax.numpy as jnp
from jax import lax
from jax.experimental import pallas as pl
from jax.experimental.pallas import tpu as pltpu

N_DEV = 32


def kernel(x, Win0, Wout0, Win1, Wout1, Win2, Wout2):
    blk, dmodel = x.shape
    B = N_DEV * blk
    CBLK = 1024

    def body(x_ref, win0_ref, wout0_ref, win1_ref, wout1_ref,
             win2_ref, wout2_ref, out_ref,
             xg_ref, pacc_ref, rs_ref, ag_ref, send_sems, recv_sems):
        d = lax.axis_index("i")
        right = lax.rem(d + 1, N_DEV)

        def rdma_hop(comm_ref, s, r):
            rdma = pltpu.make_async_remote_copy(
                src_ref=comm_ref.at[pl.ds(s * blk, blk), :],
                dst_ref=comm_ref.at[pl.ds(r * blk, blk), :],
                send_sem=send_sems.at[s],
                recv_sem=recv_sems.at[r],
                device_id=(right,),
                device_id_type=pl.DeviceIdType.MESH,
            )
            rdma.start()
            rdma.wait()

        def ring(n_hops, hop_fn):
            def pair(t, carry):
                hop_fn(2 * t, 0, 1)
                hop_fn(2 * t + 1, 1, 0)
                return carry
            lax.fori_loop(0, n_hops // 2, pair, 0)
            if n_hops % 2:
                h = n_hops - 1
                hop_fn(h, h % 2, (h + 1) % 2)

        def ag_hop(m):
            def fn(h, s, r):
                rdma_hop(ag_ref, s, r)
                origin = lax.rem(m - h - 1 + 2 * N_DEV, N_DEV)
                xg_ref[pl.ds(origin * blk, blk), :] = ag_ref[pl.ds(r * blk, blk), :]
            return fn

        def rs_hop(h, s, r):
            rdma_hop(rs_ref, s, r)
            c = lax.rem(d - h - 1 + 2 * N_DEV, N_DEV)
            cur = rs_ref[pl.ds(r * blk, blk), :]
            rs_ref[pl.ds(r * blk, blk), :] = cur + pacc_ref[pl.ds(c * blk, blk), :]

        def compute_layer(win_ref, wout_ref):
            win = win_ref[:].astype(jnp.bfloat16)
            wout = wout_ref[:].astype(jnp.bfloat16)
            for b0 in range(0, B, CBLK):
                xb = xg_ref[pl.ds(b0, CBLK), :]
                h = jnp.dot(xb, win, preferred_element_type=jnp.float32)
                h = jnp.maximum(h, 0.0).astype(jnp.bfloat16)
                pacc_ref[pl.ds(b0, CBLK), :] = jnp.dot(
                    h, wout, preferred_element_type=jnp.float32)

        myx = x_ref[:].astype(jnp.bfloat16)
        xg_ref[pl.ds(d * blk, blk), :] = myx
        ag_ref[pl.ds(0, blk), :] = myx
        ring(N_DEV - 1, ag_hop(d))

        for win_ref, wout_ref in ((win0_ref, wout0_ref),
                                  (win1_ref, wout1_ref),
                                  (win2_ref, wout2_ref)):
            compute_layer(win_ref, wout_ref)
            rs_ref[pl.ds(0, blk), :] = pacc_ref[pl.ds(d * blk, blk), :]
            ring(N_DEV - 1, rs_hop)
            m = right
            chunk = rs_ref[pl.ds(blk, blk), :].astype(jnp.bfloat16)
            xg_ref[pl.ds(m * blk, blk), :] = chunk
            ag_ref[pl.ds(0, blk), :] = chunk
            ring(N_DEV - 1, ag_hop(m))

        out_ref[:, :] = xg_ref[:, :].astype(jnp.float32)

    return pl.pallas_call(
        body,
        out_shape=jax.ShapeDtypeStruct((B, dmodel), jnp.float32),
        in_specs=[pl.BlockSpec(memory_space=pltpu.VMEM)] * 7,
        out_specs=pl.BlockSpec(memory_space=pltpu.VMEM),
        scratch_shapes=[
            pltpu.VMEM((B, dmodel), jnp.bfloat16),
            pltpu.VMEM((B, dmodel), jnp.float32),
            pltpu.VMEM((2 * blk, dmodel), jnp.float32),
            pltpu.VMEM((2 * blk, dmodel), jnp.bfloat16),
            pltpu.SemaphoreType.DMA((2,)),
            pltpu.SemaphoreType.DMA((2,)),
        ],
    )(x, Win0, Wout0, Win1, Wout1, Win2, Wout2)


# device time: 715129 ns/iter; 1.2343x vs baseline; 1.2343x over previous
import jax
import jax.numpy as jnp
from jax import lax
from jax.experimental import pallas as pl
from jax.experimental.pallas import tpu as pltpu

N_DEV = 32


def kernel(x, Win0, Wout0, Win1, Wout1, Win2, Wout2):
    blk, dmodel = x.shape
    B = N_DEV * blk
    CBLK = 1024

    def body(x_ref, win0_ref, wout0_ref, win1_ref, wout1_ref,
             win2_ref, wout2_ref, out_ref,
             xg_ref, pacc_ref, rs_cw, rs_ccw, ag_cw, ag_ccw,
             send_cw, recv_cw, send_ccw, recv_ccw):
        d = lax.axis_index("i")
        right = lax.rem(d + 1, N_DEV)
        left = lax.rem(d - 1 + N_DEV, N_DEV)

        def mod(v):
            return lax.rem(v + 2 * N_DEV, N_DEV)

        barrier_sem = pltpu.get_barrier_semaphore()
        for nbr in (left, right):
            pl.semaphore_signal(barrier_sem, inc=1, device_id=(nbr,),
                                device_id_type=pl.DeviceIdType.MESH)
        pl.semaphore_wait(barrier_sem, 2)

        def mk(buf, s, r, target, ssem, rsem):
            return pltpu.make_async_remote_copy(
                src_ref=buf.at[pl.ds(s * blk, blk), :],
                dst_ref=buf.at[pl.ds(r * blk, blk), :],
                send_sem=ssem.at[s],
                recv_sem=rsem.at[r],
                device_id=(target,),
                device_id_type=pl.DeviceIdType.MESH,
            )

        def run_bidi(buf_cw, buf_ccw, proc_cw, proc_ccw):
            def one(h, s, r, with_ccw, cw_final=False, ccw_final=False):
                r_cw = mk(buf_cw, s, r, right, send_cw, recv_cw)
                r_cw.start()
                if with_ccw:
                    r_ccw = mk(buf_ccw, s, r, left, send_ccw, recv_ccw)
                    r_ccw.start()
                r_cw.wait()
                if with_ccw:
                    r_ccw.wait()
                proc_cw(h, r, cw_final)
                if with_ccw:
                    proc_ccw(h, r, ccw_final)

            def pair(t, carry):
                one(2 * t, 0, 1, True)
                one(2 * t + 1, 1, 0, True)
                return carry
            lax.fori_loop(0, 7, pair, 0)
            one(14, 0, 1, True, ccw_final=True)
            one(15, 1, 0, False, cw_final=True)

        def ag_proc_cw(h, r, final):
            origin = mod(d - 1 - h)
            xg_ref[pl.ds(origin * blk, blk), :] = ag_cw[pl.ds(r * blk, blk), :]

        def ag_proc_ccw(h, r, final):
            origin = mod(d + 1 + h)
            xg_ref[pl.ds(origin * blk, blk), :] = ag_ccw[pl.ds(r * blk, blk), :]

        def all_gather(chunk_bf16):
            xg_ref[pl.ds(d * blk, blk), :] = chunk_bf16
            ag_cw[pl.ds(0, blk), :] = chunk_bf16
            ag_ccw[pl.ds(0, blk), :] = chunk_bf16
            run_bidi(ag_cw, ag_ccw, ag_proc_cw, ag_proc_ccw)

        def rs_proc_cw(h, r, final):
            if not final:
                c = mod(d + 15 - h)
                cur = rs_cw[pl.ds(r * blk, blk), :]
                rs_cw[pl.ds(r * blk, blk), :] = cur + pacc_ref[pl.ds(c * blk, blk), :]

        def rs_proc_ccw(h, r, final):
            if not final:
                c = mod(d - 14 + h)
                cur = rs_ccw[pl.ds(r * blk, blk), :]
                rs_ccw[pl.ds(r * blk, blk), :] = cur + pacc_ref[pl.ds(c * blk, blk), :]

        def reduce_scatter():
            rs_cw[pl.ds(0, blk), :] = pacc_ref[pl.ds(mod(d + 16) * blk, blk), :]
            rs_ccw[pl.ds(0, blk), :] = pacc_ref[pl.ds(mod(d - 15) * blk, blk), :]
            run_bidi(rs_cw, rs_ccw, rs_proc_cw, rs_proc_ccw)
            return (rs_cw[pl.ds(0, blk), :] + rs_ccw[pl.ds(blk, blk), :]
                    + pacc_ref[pl.ds(d * blk, blk), :])

        def compute_layer(win_ref, wout_ref):
            win = win_ref[:].astype(jnp.bfloat16)
            wout = wout_ref[:].astype(jnp.bfloat16)
            for b0 in range(0, B, CBLK):
                xb = xg_ref[pl.ds(b0, CBLK), :]
                h = jnp.dot(xb, win, preferred_element_type=jnp.float32)
                h = jnp.maximum(h, 0.0).astype(jnp.bfloat16)
                pacc_ref[pl.ds(b0, CBLK), :] = jnp.dot(
                    h, wout, preferred_element_type=jnp.float32)

        all_gather(x_ref[:].astype(jnp.bfloat16))

        for win_ref, wout_ref in ((win0_ref, wout0_ref),
                                  (win1_ref, wout1_ref),
                                  (win2_ref, wout2_ref)):
            compute_layer(win_ref, wout_ref)
            mychunk = reduce_scatter()
            all_gather(mychunk.astype(jnp.bfloat16))

        out_ref[:, :] = xg_ref[:, :].astype(jnp.float32)

    return pl.pallas_call(
        body,
        out_shape=jax.ShapeDtypeStruct((B, dmodel), jnp.float32),
        in_specs=[pl.BlockSpec(memory_space=pltpu.VMEM)] * 7,
        out_specs=pl.BlockSpec(memory_space=pltpu.VMEM),
        scratch_shapes=[
            pltpu.VMEM((B, dmodel), jnp.bfloat16),
            pltpu.VMEM((B, dmodel), jnp.float32),
            pltpu.VMEM((2 * blk, dmodel), jnp.float32),
            pltpu.VMEM((2 * blk, dmodel), jnp.float32),
            pltpu.VMEM((2 * blk, dmodel), jnp.bfloat16),
            pltpu.VMEM((2 * blk, dmodel), jnp.bfloat16),
            pltpu.SemaphoreType.DMA((2,)),
            pltpu.SemaphoreType.DMA((2,)),
            pltpu.SemaphoreType.DMA((2,)),
            pltpu.SemaphoreType.DMA((2,)),
        ],
        compiler_params=pltpu.CompilerParams(collective_id=0),
    )(x, Win0, Wout0, Win1, Wout1, Win2, Wout2)


# device time: 449629 ns/iter; 1.9631x vs baseline; 1.5905x over previous
import jax
import jax.numpy as jnp
import numpy as np
from jax import lax
from jax.experimental import pallas as pl
from jax.experimental.pallas import tpu as pltpu

N_DEV = 32

_CYCLE = [0, 1, 9, 8, 16, 17, 25, 24, 27, 26, 18, 19, 11, 10, 13, 12,
          20, 21, 29, 28, 31, 30, 22, 23, 15, 14, 6, 7, 4, 5, 2, 3]
_POS = [0] * N_DEV
for _p, _i in enumerate(_CYCLE):
    _POS[_i] = _p


def kernel(x, Win0, Wout0, Win1, Wout1, Win2, Wout2):
    blk, dmodel = x.shape
    B = N_DEV * blk
    CBLK = 1024

    cyc_tab = jnp.asarray(_CYCLE, dtype=jnp.int32)
    pos_tab = jnp.asarray(_POS, dtype=jnp.int32)

    def body(cyc_ref, pos_ref, x_ref, win0_ref, wout0_ref, win1_ref,
             wout1_ref, win2_ref, wout2_ref, out_ref,
             xg_ref, pacc_ref, rs_cw, rs_ccw, ag_cw, ag_ccw,
             send_cw, recv_cw, send_ccw, recv_ccw):
        d = lax.axis_index("i")

        def mod(v):
            return lax.rem(v + 2 * N_DEV, N_DEV)

        p = pos_ref[d]
        right = cyc_ref[mod(p + 1)]
        left = cyc_ref[mod(p - 1)]

        barrier_sem = pltpu.get_barrier_semaphore()
        for nbr in (left, right):
            pl.semaphore_signal(barrier_sem, inc=1, device_id=(nbr,),
                                device_id_type=pl.DeviceIdType.MESH)
        pl.semaphore_wait(barrier_sem, 2)

        def mk(buf, s, r, target, ssem, rsem):
            return pltpu.make_async_remote_copy(
                src_ref=buf.at[pl.ds(s * blk, blk), :],
                dst_ref=buf.at[pl.ds(r * blk, blk), :],
                send_sem=ssem.at[s],
                recv_sem=rsem.at[r],
                device_id=(target,),
                device_id_type=pl.DeviceIdType.MESH,
            )

        def run_bidi(buf_cw, buf_ccw, pre_cw, pre_ccw, post_cw, post_ccw):
            def one(h, s, r, with_ccw, cw_final=False, ccw_final=False):
                d_cw = mk(buf_cw, s, r, right, send_cw, recv_cw)
                d_cw.start()
                if with_ccw:
                    d_ccw = mk(buf_ccw, s, r, left, send_ccw, recv_ccw)
                    d_ccw.start()
                d_cw.wait()
                if with_ccw:
                    d_ccw.wait()
                pre_cw(h, r, cw_final)
                if with_ccw:
                    pre_ccw(h, r, ccw_final)
                post_cw(h, r, cw_final)
                if with_ccw:
                    post_ccw(h, r, ccw_final)

            def pair(t, carry):
                one(2 * t, 0, 1, True)
                one(2 * t + 1, 1, 0, True)
                return carry
            lax.fori_loop(0, 7, pair, 0)
            one(14, 0, 1, True, ccw_final=True)
            one(15, 1, 0, False, cw_final=True)

        def noop(h, r, final):
            pass

        def ag_proc_cw(h, r, final):
            b = cyc_ref[mod(p - 1 - h)]
            xg_ref[pl.ds(b * blk, blk), :] = ag_cw[pl.ds(r * blk, blk), :]

        def ag_proc_ccw(h, r, final):
            b = cyc_ref[mod(p + 1 + h)]
            xg_ref[pl.ds(b * blk, blk), :] = ag_ccw[pl.ds(r * blk, blk), :]

        def all_gather(chunk_bf16):
            xg_ref[pl.ds(d * blk, blk), :] = chunk_bf16
            ag_cw[pl.ds(0, blk), :] = chunk_bf16
            ag_ccw[pl.ds(0, blk), :] = chunk_bf16
            run_bidi(ag_cw, ag_ccw, noop, noop, ag_proc_cw, ag_proc_ccw)

        def rs_proc_cw(h, r, final):
            if not final:
                b = cyc_ref[mod(p + 15 - h)]
                cur = rs_cw[pl.ds(r * blk, blk), :]
                rs_cw[pl.ds(r * blk, blk), :] = cur + pacc_ref[pl.ds(b * blk, blk), :]

        def rs_proc_ccw(h, r, final):
            if not final:
                b = cyc_ref[mod(p - 14 + h)]
                cur = rs_ccw[pl.ds(r * blk, blk), :]
                rs_ccw[pl.ds(r * blk, blk), :] = cur + pacc_ref[pl.ds(b * blk, blk), :]

        def reduce_scatter():
            rs_cw[pl.ds(0, blk), :] = pacc_ref[pl.ds(cyc_ref[mod(p + 16)] * blk, blk), :]
            rs_ccw[pl.ds(0, blk), :] = pacc_ref[pl.ds(cyc_ref[mod(p - 15)] * blk, blk), :]
            run_bidi(rs_cw, rs_ccw, rs_proc_cw, rs_proc_ccw, noop, noop)
            return (rs_cw[pl.ds(0, blk), :] + rs_ccw[pl.ds(blk, blk), :]
                    + pacc_ref[pl.ds(d * blk, blk), :])

        def compute_layer(win_ref, wout_ref):
            win = win_ref[:].astype(jnp.bfloat16)
            wout = wout_ref[:].astype(jnp.bfloat16)
            for b0 in range(0, B, CBLK):
                xb = xg_ref[pl.ds(b0, CBLK), :]
                h = jnp.dot(xb, win, preferred_element_type=jnp.float32)
                h = jnp.maximum(h, 0.0).astype(jnp.bfloat16)
                pacc_ref[pl.ds(b0, CBLK), :] = jnp.dot(
                    h, wout, preferred_element_type=jnp.float32)

        all_gather(x_ref[:].astype(jnp.bfloat16))

        for win_ref, wout_ref in ((win0_ref, wout0_ref),
                                  (win1_ref, wout1_ref),
                                  (win2_ref, wout2_ref)):
            compute_layer(win_ref, wout_ref)
            mychunk = reduce_scatter()
            all_gather(mychunk.astype(jnp.bfloat16))

        out_ref[:, :] = xg_ref[:, :].astype(jnp.float32)

    return pl.pallas_call(
        body,
        out_shape=jax.ShapeDtypeStruct((B, dmodel), jnp.float32),
        in_specs=[pl.BlockSpec(memory_space=pltpu.SMEM)] * 2
        + [pl.BlockSpec(memory_space=pltpu.VMEM)] * 7,
        out_specs=pl.BlockSpec(memory_space=pltpu.VMEM),
        scratch_shapes=[
            pltpu.VMEM((B, dmodel), jnp.bfloat16),
            pltpu.VMEM((B, dmodel), jnp.float32),
            pltpu.VMEM((2 * blk, dmodel), jnp.float32),
            pltpu.VMEM((2 * blk, dmodel), jnp.float32),
            pltpu.VMEM((2 * blk, dmodel), jnp.bfloat16),
            pltpu.VMEM((2 * blk, dmodel), jnp.bfloat16),
            pltpu.SemaphoreType.DMA((2,)),
            pltpu.SemaphoreType.DMA((2,)),
            pltpu.SemaphoreType.DMA((2,)),
            pltpu.SemaphoreType.DMA((2,)),
        ],
        compiler_params=pltpu.CompilerParams(collective_id=0),
    )(cyc_tab, pos_tab, x, Win0, Wout0, Win1, Wout1, Win2, Wout2)


# device time: 448984 ns/iter; 1.9659x vs baseline; 1.0014x over previous
import jax
import jax.numpy as jnp
import numpy as np
from jax import lax
from jax.experimental import pallas as pl
from jax.experimental.pallas import tpu as pltpu

N_DEV = 32

_CYCLE = [0, 1, 9, 8, 16, 17, 25, 24, 27, 26, 18, 19, 11, 10, 13, 12,
          20, 21, 29, 28, 31, 30, 22, 23, 15, 14, 6, 7, 4, 5, 2, 3]
_POS = [0] * N_DEV
for _p, _i in enumerate(_CYCLE):
    _POS[_i] = _p


def kernel(x, Win0, Wout0, Win1, Wout1, Win2, Wout2):
    blk, dmodel = x.shape
    B = N_DEV * blk
    CBLK = 1024

    cyc_tab = jnp.asarray(_CYCLE, dtype=jnp.int32)
    pos_tab = jnp.asarray(_POS, dtype=jnp.int32)

    def body(cyc_ref, pos_ref, x_ref, win0_ref, wout0_ref, win1_ref,
             wout1_ref, win2_ref, wout2_ref, out_ref,
             xg_ref, pacc_ref, rs_cw, rs_ccw, ag_cw, ag_ccw,
             send_cw, recv_cw, send_ccw, recv_ccw):
        d = lax.axis_index("i")

        def mod(v):
            return lax.rem(v + 2 * N_DEV, N_DEV)

        p = pos_ref[d]
        right = cyc_ref[mod(p + 1)]
        left = cyc_ref[mod(p - 1)]

        barrier_sem = pltpu.get_barrier_semaphore()
        for nbr in (left, right):
            pl.semaphore_signal(barrier_sem, inc=1, device_id=(nbr,),
                                device_id_type=pl.DeviceIdType.MESH)
        pl.semaphore_wait(barrier_sem, 2)

        def mk(buf, s, r, target, ssem, rsem):
            return pltpu.make_async_remote_copy(
                src_ref=buf.at[pl.ds(s * blk, blk), :],
                dst_ref=buf.at[pl.ds(r * blk, blk), :],
                send_sem=ssem.at[s],
                recv_sem=rsem.at[r],
                device_id=(target,),
                device_id_type=pl.DeviceIdType.MESH,
            )

        def run_bidi(buf_cw, buf_ccw, pre_cw, pre_ccw, post_cw, post_ccw):
            def one(h, s, r, with_ccw, pend, cw_final=False, ccw_final=False):
                d_cw = mk(buf_cw, s, r, right, send_cw, recv_cw)
                d_cw.start()
                if with_ccw:
                    d_ccw = mk(buf_ccw, s, r, left, send_ccw, recv_ccw)
                    d_ccw.start()
                if pend:
                    post_cw(h - 1, s, False)
                    post_ccw(h - 1, s, False)
                d_cw.wait()
                if with_ccw:
                    d_ccw.wait()
                pre_cw(h, r, cw_final)
                if with_ccw:
                    pre_ccw(h, r, ccw_final)

            one(0, 0, 1, True, False)
            one(1, 1, 0, True, True)

            def pair(t, carry):
                one(2 * t, 0, 1, True, True)
                one(2 * t + 1, 1, 0, True, True)
                return carry
            lax.fori_loop(1, 7, pair, 0)
            one(14, 0, 1, True, True, ccw_final=True)
            one(15, 1, 0, False, True, cw_final=True)
            post_cw(15, 0, True)

        def noop(h, r, final):
            pass

        def ag_proc_cw(h, r, final):
            b = cyc_ref[mod(p - 1 - h)]
            xg_ref[pl.ds(b * blk, blk), :] = ag_cw[pl.ds(r * blk, blk), :]

        def ag_proc_ccw(h, r, final):
            b = cyc_ref[mod(p + 1 + h)]
            xg_ref[pl.ds(b * blk, blk), :] = ag_ccw[pl.ds(r * blk, blk), :]

        def all_gather(chunk_bf16):
            xg_ref[pl.ds(d * blk, blk), :] = chunk_bf16
            ag_cw[pl.ds(0, blk), :] = chunk_bf16
            ag_ccw[pl.ds(0, blk), :] = chunk_bf16
            run_bidi(ag_cw, ag_ccw, noop, noop, ag_proc_cw, ag_proc_ccw)

        def rs_proc_cw(h, r, final):
            if not final:
                b = cyc_ref[mod(p + 15 - h)]
                cur = rs_cw[pl.ds(r * blk, blk), :]
                rs_cw[pl.ds(r * blk, blk), :] = cur + pacc_ref[pl.ds(b * blk, blk), :]

        def rs_proc_ccw(h, r, final):
            if not final:
                b = cyc_ref[mod(p - 14 + h)]
                cur = rs_ccw[pl.ds(r * blk, blk), :]
                rs_ccw[pl.ds(r * blk, blk), :] = cur + pacc_ref[pl.ds(b * blk, blk), :]

        def reduce_scatter():
            rs_cw[pl.ds(0, blk), :] = pacc_ref[pl.ds(cyc_ref[mod(p + 16)] * blk, blk), :]
            rs_ccw[pl.ds(0, blk), :] = pacc_ref[pl.ds(cyc_ref[mod(p - 15)] * blk, blk), :]
            run_bidi(rs_cw, rs_ccw, rs_proc_cw, rs_proc_ccw, noop, noop)
            return (rs_cw[pl.ds(0, blk), :] + rs_ccw[pl.ds(blk, blk), :]
                    + pacc_ref[pl.ds(d * blk, blk), :])

        def compute_layer(win_ref, wout_ref):
            win = win_ref[:].astype(jnp.bfloat16)
            wout = wout_ref[:].astype(jnp.bfloat16)
            for b0 in range(0, B, CBLK):
                xb = xg_ref[pl.ds(b0, CBLK), :]
                h = jnp.dot(xb, win, preferred_element_type=jnp.float32)
                h = jnp.maximum(h, 0.0).astype(jnp.bfloat16)
                pacc_ref[pl.ds(b0, CBLK), :] = jnp.dot(
                    h, wout, preferred_element_type=jnp.float32)

        all_gather(x_ref[:].astype(jnp.bfloat16))

        for win_ref, wout_ref in ((win0_ref, wout0_ref),
                                  (win1_ref, wout1_ref),
                                  (win2_ref, wout2_ref)):
            compute_layer(win_ref, wout_ref)
            mychunk = reduce_scatter()
            all_gather(mychunk.astype(jnp.bfloat16))

        out_ref[:, :] = xg_ref[:, :].astype(jnp.float32)

    return pl.pallas_call(
        body,
        out_shape=jax.ShapeDtypeStruct((B, dmodel), jnp.float32),
        in_specs=[pl.BlockSpec(memory_space=pltpu.SMEM)] * 2
        + [pl.BlockSpec(memory_space=pltpu.VMEM)] * 7,
        out_specs=pl.BlockSpec(memory_space=pltpu.VMEM),
        scratch_shapes=[
            pltpu.VMEM((B, dmodel), jnp.bfloat16),
            pltpu.VMEM((B, dmodel), jnp.float32),
            pltpu.VMEM((2 * blk, dmodel), jnp.float32),
            pltpu.VMEM((2 * blk, dmodel), jnp.float32),
            pltpu.VMEM((2 * blk, dmodel), jnp.bfloat16),
            pltpu.VMEM((2 * blk, dmodel), jnp.bfloat16),
            pltpu.SemaphoreType.DMA((2,)),
            pltpu.SemaphoreType.DMA((2,)),
            pltpu.SemaphoreType.DMA((2,)),
            pltpu.SemaphoreType.DMA((2,)),
        ],
        compiler_params=pltpu.CompilerParams(collective_id=0),
    )(cyc_tab, pos_tab, x, Win0, Wout0, Win1, Wout1, Win2, Wout2)
